# baseline (speedup 1.0000x reference)
import functools

import jax
import jax.numpy as jnp
from jax import lax
from jax.experimental import pallas as pl
from jax.experimental.pallas import tpu as pltpu
from jax.experimental.pallas import tpu_sc as plsc

_NB = 16
_ALPHA = 0.1
_LEVELS = float((1 << _NB) - 1)
_D0 = 16384
_D1 = 200


@functools.lru_cache(maxsize=None)
def _make_sc_kernel():
    info = plsc.get_sparse_core_info()
    nc, ns, L = info.num_cores, info.num_subcores, info.num_lanes
    nw = nc * ns
    tcols = _D0 // 128 // nw
    cols = tcols * 128
    run = cols * 8 // 128
    rows_per_d1 = _D0 * _NB // 128
    half_rows = rows_per_d1 // 2
    dgrp = 4
    ngrp = _D1 // dgrp
    mesh = plsc.VectorSubcoreMesh(core_axis_name="c", subcore_axis_name="s")

    @functools.partial(
        pl.kernel,
        mesh=mesh,
        out_type=jax.ShapeDtypeStruct((_D1 * _D0 * _NB // 128, 128), jnp.float32),
        scratch_types=[
            pltpu.VMEM((2, dgrp, tcols, 128), jnp.float32),
            pltpu.VMEM((dgrp, 2 * run, 128), jnp.float32),
            pltpu.SemaphoreType.DMA,
            pltpu.SemaphoreType.DMA,
            pltpu.SemaphoreType.DMA,
            pltpu.SemaphoreType.DMA,
            pltpu.SemaphoreType.DMA,
            pltpu.SemaphoreType.DMA,
        ],
        compiler_params=pltpu.CompilerParams(
            needs_layout_passes=False, use_tc_tiling_on_sc=True
        ),
    )
    def gc_kernel(x_hbm, out_hbm, x_v, out_v, si0, si1, so0, so1, so2, so3):
        wid = lax.axis_index("s") * nc + lax.axis_index("c")
        sems_in = (si0, si1)
        sems_out = (so0, so1, so2, so3)

        def x_src(g4):
            return x_hbm.at[pl.ds(g4 * dgrp, dgrp), pl.ds(wid * tcols, tcols), :]

        pltpu.async_copy(x_src(0), x_v.at[0], sems_in[0])
        pltpu.async_copy(x_src(1), x_v.at[1], sems_in[1])

        def group(g4, pin):
            pltpu.make_async_copy(x_src(g4), x_v.at[pin], sems_in[pin]).wait()
            for dl in range(dgrp):
                d1 = g4 * dgrp + dl
                row0 = d1 * rows_per_d1 + wid * run

                @pl.when(g4 >= 1)
                def _wait_out():
                    prev_row0 = (d1 - dgrp) * rows_per_d1 + wid * run
                    pltpu.make_async_copy(
                        out_v.at[dl, pl.ds(0, run)],
                        out_hbm.at[pl.ds(prev_row0, run)], sems_out[dl],
                    ).wait()
                    pltpu.make_async_copy(
                        out_v.at[dl, pl.ds(run, run)],
                        out_hbm.at[pl.ds(prev_row0 + half_rows, run)],
                        sems_out[dl],
                    ).wait()

                def blk(b, carry2):
                    xv = x_v[pin, dl, b // 8, pl.ds((b % 8) * L, L)]
                    q = (xv * _LEVELS).astype(jnp.int32)
                    g = jnp.bitwise_xor(q, jnp.right_shift(q, 1))
                    scale = xv * _ALPHA + 1.0
                    zeros = jnp.zeros((L,), jnp.float32)
                    for i in range(_NB):
                        m = jnp.bitwise_and(g, (1 << i)) != 0
                        val = jnp.where(m, scale, zeros)
                        row = (i // 8) * run + (b // 8) * 8 + (i % 8)
                        out_v[dl, row, pl.ds((b % 8) * L, L)] = val
                    return carry2

                lax.fori_loop(0, cols // L, blk, 0, unroll=3)

                pltpu.async_copy(
                    out_v.at[dl, pl.ds(0, run)], out_hbm.at[pl.ds(row0, run)],
                    sems_out[dl],
                )
                pltpu.async_copy(
                    out_v.at[dl, pl.ds(run, run)],
                    out_hbm.at[pl.ds(row0 + half_rows, run)], sems_out[dl],
                )

            @pl.when(g4 + 2 < ngrp)
            def _prefetch():
                pltpu.async_copy(x_src(g4 + 2), x_v.at[pin], sems_in[pin])

        def outer(gg, carry):
            group(gg * 2, 0)
            group(gg * 2 + 1, 1)
            return carry

        lax.fori_loop(0, ngrp // 2, outer, 0)

        for dl in range(dgrp):
            d1 = _D1 - dgrp + dl
            row0 = d1 * rows_per_d1 + wid * run
            pltpu.make_async_copy(
                out_v.at[dl, pl.ds(0, run)], out_hbm.at[pl.ds(row0, run)],
                sems_out[dl],
            ).wait()
            pltpu.make_async_copy(
                out_v.at[dl, pl.ds(run, run)],
                out_hbm.at[pl.ds(row0 + half_rows, run)], sems_out[dl],
            ).wait()

    return gc_kernel


def kernel(x):
    x3 = jnp.transpose(x.reshape(128, 128, _D1), (2, 0, 1))
    out_lin = _make_sc_kernel()(x3)
    out6 = out_lin.reshape(_D1, 2, _D0 // 128, 8, 128)
    return jnp.transpose(out6, (2, 4, 0, 1, 3)).reshape(_D0, _D1, _NB)


# speedup vs baseline: 1.1311x; 1.1311x over previous
import functools

import jax
import jax.numpy as jnp
from jax import lax
from jax.experimental import pallas as pl
from jax.experimental.pallas import tpu as pltpu
from jax.experimental.pallas import tpu_sc as plsc

_NB = 16
_ALPHA = 0.1
_LEVELS = float((1 << _NB) - 1)
_D0 = 16384
_D1 = 200


@functools.lru_cache(maxsize=None)
def _make_sc_kernel():
    info = plsc.get_sparse_core_info()
    nc, ns, L = info.num_cores, info.num_subcores, info.num_lanes
    nw = nc * ns
    tcols = _D0 // 128 // nw
    cols = tcols * 128
    run = cols * 8 // 128
    rows_per_d1 = _D0 * _NB // 128
    half_rows = rows_per_d1 // 2
    dgrp = 8
    ngrp = _D1 // dgrp
    mesh = plsc.VectorSubcoreMesh(core_axis_name="c", subcore_axis_name="s")

    @functools.partial(
        pl.kernel,
        mesh=mesh,
        out_type=jax.ShapeDtypeStruct((_D1 * _D0 * _NB // 128, 128), jnp.float32),
        scratch_types=[
            pltpu.VMEM((2, dgrp, tcols, 128), jnp.float32),
            pltpu.VMEM((dgrp, 2 * run, 128), jnp.float32),
            pltpu.SemaphoreType.DMA,
            pltpu.SemaphoreType.DMA,
            pltpu.SemaphoreType.DMA,
            pltpu.SemaphoreType.DMA,
            pltpu.SemaphoreType.DMA,
            pltpu.SemaphoreType.DMA,
            pltpu.SemaphoreType.DMA,
            pltpu.SemaphoreType.DMA,
            pltpu.SemaphoreType.DMA,
            pltpu.SemaphoreType.DMA,
        ],
        compiler_params=pltpu.CompilerParams(
            needs_layout_passes=False, use_tc_tiling_on_sc=True
        ),
    )
    def gc_kernel(
        x_hbm, out_hbm, x_v, out_v, si0, si1, so0, so1, so2, so3, so4, so5,
        so6, so7,
    ):
        wid = lax.axis_index("s") * nc + lax.axis_index("c")
        sems_in = (si0, si1)
        sems_out = (so0, so1, so2, so3, so4, so5, so6, so7)

        def x_src(g4):
            return x_hbm.at[pl.ds(g4 * dgrp, dgrp), pl.ds(wid * tcols, tcols), :]

        pltpu.async_copy(x_src(0), x_v.at[0], sems_in[0])
        pltpu.async_copy(x_src(1), x_v.at[1], sems_in[1])

        def group(g4, pin):
            pltpu.make_async_copy(x_src(g4), x_v.at[pin], sems_in[pin]).wait()
            for dl in range(dgrp):
                d1 = g4 * dgrp + dl
                row0 = d1 * rows_per_d1 + wid * run

                @pl.when(g4 >= 1)
                def _wait_out():
                    prev_row0 = (d1 - dgrp) * rows_per_d1 + wid * run
                    pltpu.make_async_copy(
                        out_v.at[dl, pl.ds(0, run)],
                        out_hbm.at[pl.ds(prev_row0, run)], sems_out[dl],
                    ).wait()
                    pltpu.make_async_copy(
                        out_v.at[dl, pl.ds(run, run)],
                        out_hbm.at[pl.ds(prev_row0 + half_rows, run)],
                        sems_out[dl],
                    ).wait()

                def blk(b, carry2):
                    xv = x_v[pin, dl, b // 8, pl.ds((b % 8) * L, L)]
                    q = (xv * _LEVELS).astype(jnp.int32)
                    g = jnp.bitwise_xor(q, jnp.right_shift(q, 1))
                    scale = xv * _ALPHA + 1.0
                    zeros = jnp.zeros((L,), jnp.float32)
                    for i in range(_NB):
                        m = jnp.bitwise_and(g, (1 << i)) != 0
                        val = jnp.where(m, scale, zeros)
                        row = (i // 8) * run + (b // 8) * 8 + (i % 8)
                        out_v[dl, row, pl.ds((b % 8) * L, L)] = val
                    return carry2

                lax.fori_loop(0, cols // L, blk, 0, unroll=2)

                pltpu.async_copy(
                    out_v.at[dl, pl.ds(0, run)], out_hbm.at[pl.ds(row0, run)],
                    sems_out[dl],
                )
                pltpu.async_copy(
                    out_v.at[dl, pl.ds(run, run)],
                    out_hbm.at[pl.ds(row0 + half_rows, run)], sems_out[dl],
                )

            @pl.when(g4 + 2 < ngrp)
            def _prefetch():
                pltpu.async_copy(x_src(g4 + 2), x_v.at[pin], sems_in[pin])

        def outer(gg, carry):
            group(gg * 2, 0)
            group(gg * 2 + 1, 1)
            return carry

        lax.fori_loop(0, ngrp // 2, outer, 0)

        for dl in range(dgrp):
            d1 = _D1 - dgrp + dl
            row0 = d1 * rows_per_d1 + wid * run
            pltpu.make_async_copy(
                out_v.at[dl, pl.ds(0, run)], out_hbm.at[pl.ds(row0, run)],
                sems_out[dl],
            ).wait()
            pltpu.make_async_copy(
                out_v.at[dl, pl.ds(run, run)],
                out_hbm.at[pl.ds(row0 + half_rows, run)], sems_out[dl],
            ).wait()

    return gc_kernel


def kernel(x):
    x3 = jnp.transpose(x.reshape(128, 128, _D1), (2, 0, 1))
    out_lin = _make_sc_kernel()(x3)
    out6 = out_lin.reshape(_D1, 2, _D0 // 128, 8, 128)
    return jnp.transpose(out6, (2, 4, 0, 1, 3)).reshape(_D0, _D1, _NB)
